# Initial kernel scaffold; baseline (speedup 1.0000x reference)
import jax
import jax.numpy as jnp
from jax.experimental import pallas as pl


def kernel(x, edge_index, batch, global_features, W_emb, b_emb, W0, a_src0, a_dst0, b0, W1, a_src1, a_dst1, b1, W2, a_src2, a_dst2, b2, W_glob, b_glob, q1_W, q1_b, q2_W, q2_b, q3_W, q3_b):
    raise NotImplementedError("write your pallas kernel here")


# speedup vs baseline: 30.9689x; 30.9689x over previous
import functools

import jax
import jax.numpy as jnp
from jax import lax
from jax.experimental import pallas as pl
from jax.experimental.pallas import tpu as pltpu
from jax.experimental.pallas import tpu_sc as plsc

N = 50000
E = 800000
G = 16
NC = 2
NS = 16
LANES = 16
RPT = 1568
R = NS * RPT
N_PAD = NC * R
RB = 512
NBLK = N_PAD // RB
EPT = E // NS
B = 80
NCH = EPT // B
R4 = R // 4
RPT4 = RPT // 4

_f32 = jnp.float32


def _elu(z):
    return jnp.where(z > 0, z, jnp.exp(z) - 1.0)


def _prep0_body(x_ref, wemb_ref, bemb_ref, w_ref, aa_ref, hs_ref, ed_ref):
    h = _elu(jnp.dot(x_ref[...], wemb_ref[...],
                     preferred_element_type=_f32) + bemb_ref[...])
    hw = jnp.dot(h, w_ref[...], preferred_element_type=_f32)
    esed = jnp.dot(hw, aa_ref[...], preferred_element_type=_f32)
    hs_ref[...] = jnp.concatenate(
        [hw, esed[:, 0:4], jnp.zeros((RB, 12), _f32)], axis=1)
    ed_ref[...] = jnp.concatenate(
        [esed[:, 4:8], jnp.zeros((RB, 12), _f32)], axis=1)


_prep0 = pl.pallas_call(
    _prep0_body,
    grid=(NBLK,),
    in_specs=[
        pl.BlockSpec((RB, 128), lambda i: (i, 0)),
        pl.BlockSpec((128, 64), lambda i: (0, 0)),
        pl.BlockSpec((1, 64), lambda i: (0, 0)),
        pl.BlockSpec((64, 64), lambda i: (0, 0)),
        pl.BlockSpec((64, 8), lambda i: (0, 0)),
    ],
    out_specs=[
        pl.BlockSpec((RB, 80), lambda i: (i, 0)),
        pl.BlockSpec((RB, 16), lambda i: (i, 0)),
    ],
    out_shape=[
        jax.ShapeDtypeStruct((N_PAD, 80), _f32),
        jax.ShapeDtypeStruct((N_PAD, 16), _f32),
    ],
)


def _prepl_body(acc_ref, den_ref, r4_ref, bprev_ref, w_ref, aa_ref,
                hs_ref, ed_ref):
    num = acc_ref[...]
    den = jnp.dot(den_ref[...], r4_ref[...], preferred_element_type=_f32)
    h = _elu(num / (den + 1e-16) + bprev_ref[...])
    hw = jnp.dot(h, w_ref[...], preferred_element_type=_f32)
    esed = jnp.dot(hw, aa_ref[...], preferred_element_type=_f32)
    hs_ref[...] = jnp.concatenate(
        [hw, esed[:, 0:4], jnp.zeros((RB, 12), _f32)], axis=1)
    ed_ref[...] = jnp.concatenate(
        [esed[:, 4:8], jnp.zeros((RB, 12), _f32)], axis=1)


_prepl = pl.pallas_call(
    _prepl_body,
    grid=(NBLK,),
    in_specs=[
        pl.BlockSpec((RB, 64), lambda i: (i, 0)),
        pl.BlockSpec((RB, 4), lambda i: (i, 0)),
        pl.BlockSpec((4, 64), lambda i: (0, 0)),
        pl.BlockSpec((1, 64), lambda i: (0, 0)),
        pl.BlockSpec((64, 64), lambda i: (0, 0)),
        pl.BlockSpec((64, 8), lambda i: (0, 0)),
    ],
    out_specs=[
        pl.BlockSpec((RB, 80), lambda i: (i, 0)),
        pl.BlockSpec((RB, 16), lambda i: (i, 0)),
    ],
    out_shape=[
        jax.ShapeDtypeStruct((N_PAD, 80), _f32),
        jax.ShapeDtypeStruct((N_PAD, 16), _f32),
    ],
)


def _sc_gat_body(hs_hbm, ed_hbm, src_hbm, dst_hbm, z_hbm, z2_hbm,
                 out_hbm, out2_hbm,
                 acc, acc2, sidx, didx, dloc, dloc2, hsb, edb, contrib,
                 contrib2, wbuf, sem1, sem2):
    c = lax.axis_index("c")
    s = lax.axis_index("s")
    base = c * R

    pltpu.sync_copy(z_hbm, acc.at[pl.ds(s * RPT, RPT)])
    pltpu.sync_copy(z2_hbm, acc2.at[pl.ds(s * RPT4, RPT4)])
    plsc.subcore_barrier()

    lanes = lax.iota(jnp.int32, 16)
    kmask = jnp.where(lanes < 4, 1.0, 0.0)

    def _chunk(ch, _):
        off = s * EPT + ch * B
        pltpu.sync_copy(src_hbm.at[pl.ds(off, B)], sidx)
        pltpu.sync_copy(dst_hbm.at[pl.ds(off, B)], didx)
        d1 = pltpu.async_copy(hs_hbm.at[sidx], hsb, sem1)
        d2 = pltpu.async_copy(ed_hbm.at[didx], edb, sem2)
        d1.wait()
        d2.wait()

        for j in range(B // 16):
            dv = didx[pl.ds(16 * j, 16)]
            own = (dv >= base) & (dv < base + R)
            dl = jnp.where(own, dv - base, 0)
            dloc[pl.ds(16 * j, 16)] = dl
            dloc2[pl.ds(16 * j, 16)] = lax.shift_right_logical(dl, 2)
            wbuf[pl.ds(16 * j, 16)] = jnp.where(own, 1.0, 0.0)

        def _grp(j, _):
            j16 = pl.multiple_of(j * 16, 16)
            mfv = wbuf[pl.ds(j16, 16)]
            dlv = dloc[pl.ds(j16, 16)]
            for jj in range(16):
                b = j16 + jj
                ev = hsb[b, pl.ds(64, 16)] + edb[b, pl.ds(0, 16)]
                ev = jnp.where(ev >= 0, ev, 0.2 * ev)
                wv = jnp.exp(ev) * (kmask * mfv[jj])
                for k in range(4):
                    contrib[b, pl.ds(16 * k, 16)] = (
                        hsb[b, pl.ds(16 * k, 16)] * wv[k])
                m4 = (dlv[jj] & 3) * 4
                den = jnp.where(lanes == m4, wv[0], 0.0)
                den = jnp.where(lanes == m4 + 1, wv[1], den)
                den = jnp.where(lanes == m4 + 2, wv[2], den)
                den = jnp.where(lanes == m4 + 3, wv[3], den)
                contrib2[b, pl.ds(0, 16)] = den
            return 0
        lax.fori_loop(0, B // 16, _grp, 0)

        pltpu.sync_copy(contrib, acc.at[dloc], add=True)
        pltpu.sync_copy(contrib2, acc2.at[dloc2], add=True)
        return 0

    lax.fori_loop(0, NCH, _chunk, 0)

    plsc.subcore_barrier()
    pltpu.sync_copy(acc.at[pl.ds(s * RPT, RPT)],
                    out_hbm.at[pl.ds(base + s * RPT, RPT)])
    pltpu.sync_copy(acc2.at[pl.ds(s * RPT4, RPT4)],
                    out2_hbm.at[pl.ds(c * R4 + s * RPT4, RPT4)])


_sc_gat = functools.partial(
    pl.kernel,
    out_type=(jax.ShapeDtypeStruct((N_PAD, 64), _f32),
              jax.ShapeDtypeStruct((N_PAD // 4, 16), _f32)),
    mesh=plsc.VectorSubcoreMesh(
        core_axis_name="c", subcore_axis_name="s",
        num_cores=NC, num_subcores=NS),
    compiler_params=pltpu.CompilerParams(use_tc_tiling_on_sc=False),
    scratch_types=[
        pltpu.VMEM_SHARED((R, 64), _f32),
        pltpu.VMEM_SHARED((R4, 16), _f32),
        pltpu.VMEM((B,), jnp.int32),
        pltpu.VMEM((B,), jnp.int32),
        pltpu.VMEM((B,), jnp.int32),
        pltpu.VMEM((B,), jnp.int32),
        pltpu.VMEM((B, 80), _f32),
        pltpu.VMEM((B, 16), _f32),
        pltpu.VMEM((B, 64), _f32),
        pltpu.VMEM((B, 16), _f32),
        pltpu.VMEM((B,), _f32),
        pltpu.SemaphoreType.DMA,
        pltpu.SemaphoreType.DMA,
    ],
)(_sc_gat_body)


def _pool_body(acc_ref, den_ref, r4_ref, b2_ref, pb_ref, g16_ref, gf_ref,
               wg_ref,
               bg_ref, q1a_ref, q1b_ref, q1c_ref, q1b_b_ref, q2w_ref,
               q2b_ref, q3w_ref, q3b_ref, out_ref,
               sums, cnts, maxs):
    i = pl.program_id(0)

    @pl.when(i == 0)
    def _init():
        sums[...] = jnp.zeros((G, 64), _f32)
        cnts[...] = jnp.zeros((G, 64), _f32)
        maxs[...] = jnp.full((G, 64), -jnp.inf, _f32)

    num = acc_ref[...]
    den = jnp.dot(den_ref[...], r4_ref[...], preferred_element_type=_f32)
    h = _elu(num / (den + 1e-16) + b2_ref[...])

    bcol = jnp.dot(pb_ref[...], g16_ref[...], preferred_element_type=_f32)
    onef = jnp.ones((RB, 64), _f32)
    for g in range(G):
        m = bcol == float(g + 1)
        sums[g:g + 1, :] += jnp.sum(jnp.where(m, h, 0.0), axis=0,
                                    keepdims=True)
        cnts[g:g + 1, :] += jnp.sum(jnp.where(m, onef, 0.0), axis=0,
                                    keepdims=True)
        mg = jnp.max(jnp.where(m, h, -jnp.inf), axis=0, keepdims=True)
        maxs[g:g + 1, :] = jnp.maximum(maxs[g:g + 1, :], mg)

    @pl.when(i == NBLK - 1)
    def _fin():
        cnt = cnts[...]
        x_mean = sums[...] / jnp.maximum(cnt, 1.0)
        x_max = jnp.where(cnt > 0, maxs[...], 0.0)
        gf = jnp.maximum(jnp.dot(gf_ref[...], wg_ref[...],
                                 preferred_element_type=_f32)
                         + bg_ref[...], 0.0)
        q = (jnp.dot(x_mean, q1a_ref[...], preferred_element_type=_f32)
             + jnp.dot(x_max, q1b_ref[...], preferred_element_type=_f32)
             + jnp.dot(gf, q1c_ref[...], preferred_element_type=_f32)
             + q1b_b_ref[...])
        q = jnp.maximum(q, 0.0)
        q = jnp.maximum(jnp.dot(q, q2w_ref[...],
                                preferred_element_type=_f32) + q2b_ref[...],
                        0.0)
        out_ref[...] = (jnp.dot(q, q3w_ref[...], preferred_element_type=_f32)
                        + q3b_ref[...])


_pool = pl.pallas_call(
    _pool_body,
    grid=(NBLK,),
    in_specs=[
        pl.BlockSpec((RB, 64), lambda i: (i, 0)),
        pl.BlockSpec((RB, 4), lambda i: (i, 0)),
        pl.BlockSpec((4, 64), lambda i: (0, 0)),
        pl.BlockSpec((1, 64), lambda i: (0, 0)),
        pl.BlockSpec((RB, 16), lambda i: (i, 0)),
        pl.BlockSpec((16, 64), lambda i: (0, 0)),
        pl.BlockSpec((16, 128), lambda i: (0, 0)),
        pl.BlockSpec((128, 64), lambda i: (0, 0)),
        pl.BlockSpec((1, 64), lambda i: (0, 0)),
        pl.BlockSpec((64, 128), lambda i: (0, 0)),
        pl.BlockSpec((64, 128), lambda i: (0, 0)),
        pl.BlockSpec((64, 128), lambda i: (0, 0)),
        pl.BlockSpec((1, 128), lambda i: (0, 0)),
        pl.BlockSpec((128, 64), lambda i: (0, 0)),
        pl.BlockSpec((1, 64), lambda i: (0, 0)),
        pl.BlockSpec((64, 32), lambda i: (0, 0)),
        pl.BlockSpec((1, 32), lambda i: (0, 0)),
    ],
    out_specs=pl.BlockSpec((G, 32), lambda i: (0, 0)),
    out_shape=jax.ShapeDtypeStruct((G, 32), _f32),
    scratch_shapes=[
        pltpu.VMEM((G, 64), _f32),
        pltpu.VMEM((G, 64), _f32),
        pltpu.VMEM((G, 64), _f32),
    ],
)


def _att_mat(a_s, a_d):
    eye = jnp.eye(4, dtype=_f32)
    a_s = (a_s[:, :, None] * eye[:, None, :]).reshape(64, 4)
    a_d = (a_d[:, :, None] * eye[:, None, :]).reshape(64, 4)
    return jnp.concatenate([a_s, a_d], axis=1)


def kernel(x, edge_index, batch, global_features, W_emb, b_emb,
           W0, a_src0, a_dst0, b0, W1, a_src1, a_dst1, b1,
           W2, a_src2, a_dst2, b2, W_glob, b_glob,
           q1_W, q1_b, q2_W, q2_b, q3_W, q3_b):
    x_pad = jnp.pad(x, ((0, N_PAD - N), (0, 0)))
    src = edge_index[0]
    dst = edge_index[1]

    r4 = jnp.repeat(jnp.eye(4, dtype=_f32), 16, axis=1)
    pb = (jnp.pad(batch, (0, N_PAD - N), constant_values=G)[:, None]
          == jnp.arange(G)[None, :]).astype(_f32)
    g16 = jnp.broadcast_to(jnp.arange(1, G + 1, dtype=_f32)[:, None], (G, 64))
    gfp = jnp.pad(global_features, ((0, 0), (0, 125)))
    wgp = jnp.pad(W_glob, ((0, 125), (0, 0)))
    q3wp = jnp.pad(q3_W, ((0, 0), (0, 2)))
    q3bp = jnp.pad(q3_b, (0, 2))

    z = jnp.zeros((RPT, 64), _f32)
    z2 = jnp.zeros((RPT4, 16), _f32)

    hs, ed = _prep0(x_pad, W_emb, b_emb.reshape(1, 64), W0,
                    _att_mat(a_src0, a_dst0))
    acc, den2 = _sc_gat(hs, ed, src, dst, z, z2)
    hs, ed = _prepl(acc, den2.reshape(N_PAD, 4), r4, b0.reshape(1, 64), W1,
                    _att_mat(a_src1, a_dst1))
    acc, den2 = _sc_gat(hs, ed, src, dst, z, z2)
    hs, ed = _prepl(acc, den2.reshape(N_PAD, 4), r4, b1.reshape(1, 64), W2,
                    _att_mat(a_src2, a_dst2))
    acc, den2 = _sc_gat(hs, ed, src, dst, z, z2)

    q = _pool(acc, den2.reshape(N_PAD, 4), r4, b2.reshape(1, 64), pb, g16,
              gfp, wgp,
              b_glob.reshape(1, 64), q1_W[0:64], q1_W[64:128], q1_W[128:192],
              q1_b.reshape(1, 128), q2_W, q2_b.reshape(1, 64), q3wp,
              q3bp.reshape(1, 32))
    return q[:, 0:30]


# speedup vs baseline: 37.4103x; 1.2080x over previous
import functools

import jax
import jax.numpy as jnp
from jax import lax
from jax.experimental import pallas as pl
from jax.experimental.pallas import tpu as pltpu
from jax.experimental.pallas import tpu_sc as plsc

N = 50000
E = 800000
G = 16
NC = 2
NS = 16
LANES = 16
RPT = 1568
R = NS * RPT
N_PAD = NC * R
RB = 512
NBLK = N_PAD // RB
EPT = E // NS
B = 80
NCH = EPT // B
R4 = R // 4
RPT4 = RPT // 4

_f32 = jnp.float32


def _elu(z):
    return jnp.where(z > 0, z, jnp.exp(z) - 1.0)


def _prep0_body(x_ref, wemb_ref, bemb_ref, w_ref, aa_ref, hs_ref, ed_ref):
    h = _elu(jnp.dot(x_ref[...], wemb_ref[...],
                     preferred_element_type=_f32) + bemb_ref[...])
    hw = jnp.dot(h, w_ref[...], preferred_element_type=_f32)
    esed = jnp.dot(hw, aa_ref[...], preferred_element_type=_f32)
    hs_ref[...] = jnp.concatenate(
        [hw, esed[:, 0:4], jnp.zeros((RB, 12), _f32)], axis=1)
    ed_ref[...] = jnp.concatenate(
        [esed[:, 4:8], jnp.zeros((RB, 12), _f32)], axis=1)


_prep0 = pl.pallas_call(
    _prep0_body,
    grid=(NBLK,),
    in_specs=[
        pl.BlockSpec((RB, 128), lambda i: (i, 0)),
        pl.BlockSpec((128, 64), lambda i: (0, 0)),
        pl.BlockSpec((1, 64), lambda i: (0, 0)),
        pl.BlockSpec((64, 64), lambda i: (0, 0)),
        pl.BlockSpec((64, 8), lambda i: (0, 0)),
    ],
    out_specs=[
        pl.BlockSpec((RB, 80), lambda i: (i, 0)),
        pl.BlockSpec((RB, 16), lambda i: (i, 0)),
    ],
    out_shape=[
        jax.ShapeDtypeStruct((N_PAD, 80), _f32),
        jax.ShapeDtypeStruct((N_PAD, 16), _f32),
    ],
)


def _prepl_body(acc_ref, den_ref, r4_ref, bprev_ref, w_ref, aa_ref,
                hs_ref, ed_ref):
    num = acc_ref[...]
    den = jnp.dot(den_ref[...], r4_ref[...], preferred_element_type=_f32)
    h = _elu(num / (den + 1e-16) + bprev_ref[...])
    hw = jnp.dot(h, w_ref[...], preferred_element_type=_f32)
    esed = jnp.dot(hw, aa_ref[...], preferred_element_type=_f32)
    hs_ref[...] = jnp.concatenate(
        [hw, esed[:, 0:4], jnp.zeros((RB, 12), _f32)], axis=1)
    ed_ref[...] = jnp.concatenate(
        [esed[:, 4:8], jnp.zeros((RB, 12), _f32)], axis=1)


_prepl = pl.pallas_call(
    _prepl_body,
    grid=(NBLK,),
    in_specs=[
        pl.BlockSpec((RB, 64), lambda i: (i, 0)),
        pl.BlockSpec((RB, 4), lambda i: (i, 0)),
        pl.BlockSpec((4, 64), lambda i: (0, 0)),
        pl.BlockSpec((1, 64), lambda i: (0, 0)),
        pl.BlockSpec((64, 64), lambda i: (0, 0)),
        pl.BlockSpec((64, 8), lambda i: (0, 0)),
    ],
    out_specs=[
        pl.BlockSpec((RB, 80), lambda i: (i, 0)),
        pl.BlockSpec((RB, 16), lambda i: (i, 0)),
    ],
    out_shape=[
        jax.ShapeDtypeStruct((N_PAD, 80), _f32),
        jax.ShapeDtypeStruct((N_PAD, 16), _f32),
    ],
)


def _sc_gat_body(hs_hbm, ed_hbm, src_hbm, dst_hbm, z_hbm, z2_hbm,
         out_hbm, out2_hbm,
         acc, acc2, sidx0, didx0, sidx1, didx1, dloc, dloc2,
         hsb0, edb0, hsb1, edb1, contrib, contrib2, wbuf,
         semh0, seme0, semh1, seme1):
    c = lax.axis_index("c"); s = lax.axis_index("s")
    base = c * R
    pltpu.sync_copy(z_hbm, acc.at[pl.ds(s * RPT, RPT)])
    pltpu.sync_copy(z2_hbm, acc2.at[pl.ds(s * RPT4, RPT4)])
    plsc.subcore_barrier()
    lanes = lax.iota(jnp.int32, 16)
    kmask = jnp.where(lanes < 4, 1.0, 0.0)
    bufs = ((sidx0, didx0, hsb0, edb0, semh0, seme0),
            (sidx1, didx1, hsb1, edb1, semh1, seme1))

    def issue(ch, bi):
        sidx, didx, hsb, edb, sh, se = bufs[bi]
        off = s * EPT + ch * B
        pltpu.sync_copy(src_hbm.at[pl.ds(off, B)], sidx)
        pltpu.sync_copy(dst_hbm.at[pl.ds(off, B)], didx)
        pltpu.async_copy(hs_hbm.at[sidx], hsb, sh)
        pltpu.async_copy(ed_hbm.at[didx], edb, se)

    def wait(bi):
        sidx, didx, hsb, edb, sh, se = bufs[bi]
        pltpu.make_async_copy(hs_hbm.at[sidx], hsb, sh).wait()
        pltpu.make_async_copy(ed_hbm.at[didx], edb, se).wait()

    def compute(bi):
        sidx, didx, hsb, edb, sh, se = bufs[bi]
        for j in range(B // 16):
            dv = didx[pl.ds(16 * j, 16)]
            own = (dv >= base) & (dv < base + R)
            dl = jnp.where(own, dv - base, 0)
            dloc[pl.ds(16 * j, 16)] = dl
            dloc2[pl.ds(16 * j, 16)] = lax.shift_right_logical(dl, 2)
            wbuf[pl.ds(16 * j, 16)] = jnp.where(own, 1.0, 0.0)
        def _grp(j, _):
            j16 = pl.multiple_of(j * 16, 16)
            mfv = wbuf[pl.ds(j16, 16)]
            dlv = dloc[pl.ds(j16, 16)]
            for jj in range(16):
                b = j16 + jj
                ev = hsb[b, pl.ds(64, 16)] + edb[b, pl.ds(0, 16)]
                ev = jnp.where(ev >= 0, ev, 0.2 * ev)
                wv = jnp.exp(ev) * (kmask * mfv[jj])
                for k in range(4):
                    contrib[b, pl.ds(16 * k, 16)] = (
                        hsb[b, pl.ds(16 * k, 16)] * wv[k])
                m4 = (dlv[jj] & 3) * 4
                den = jnp.where(lanes == m4, wv[0], 0.0)
                den = jnp.where(lanes == m4 + 1, wv[1], den)
                den = jnp.where(lanes == m4 + 2, wv[2], den)
                den = jnp.where(lanes == m4 + 3, wv[3], den)
                contrib2[b, pl.ds(0, 16)] = den
            return 0
        lax.fori_loop(0, B // 16, _grp, 0)
        pltpu.sync_copy(contrib, acc.at[dloc], add=True)
        pltpu.sync_copy(contrib2, acc2.at[dloc2], add=True)

    issue(0, 0)
    def pair(i, _):
        wait(0)
        issue(2 * i + 1, 1)
        compute(0)
        wait(1)
        issue(2 * i + 2, 0)
        compute(1)
        return 0
    lax.fori_loop(0, (NCH - 1) // 2, pair, 0)
    wait(0)
    compute(0)
    plsc.subcore_barrier()
    pltpu.sync_copy(acc.at[pl.ds(s * RPT, RPT)],
                    out_hbm.at[pl.ds(base + s * RPT, RPT)])
    pltpu.sync_copy(acc2.at[pl.ds(s * RPT4, RPT4)],
                    out2_hbm.at[pl.ds(c * R4 + s * RPT4, RPT4)])

_sc_gat = functools.partial(
    pl.kernel,
    out_type=(jax.ShapeDtypeStruct((N_PAD, 64), _f32),
              jax.ShapeDtypeStruct((N_PAD // 4, 16), _f32)),
    mesh=plsc.VectorSubcoreMesh(
        core_axis_name="c", subcore_axis_name="s",
        num_cores=NC, num_subcores=NS),
    compiler_params=pltpu.CompilerParams(use_tc_tiling_on_sc=False),
        scratch_types=[
            pltpu.VMEM_SHARED((R, 64), _f32),
            pltpu.VMEM_SHARED((R4, 16), _f32),
            pltpu.VMEM((B,), jnp.int32),
            pltpu.VMEM((B,), jnp.int32),
            pltpu.VMEM((B,), jnp.int32),
            pltpu.VMEM((B,), jnp.int32),
            pltpu.VMEM((B,), jnp.int32),
            pltpu.VMEM((B,), jnp.int32),
            pltpu.VMEM((B, 80), _f32),
            pltpu.VMEM((B, 16), _f32),
            pltpu.VMEM((B, 80), _f32),
            pltpu.VMEM((B, 16), _f32),
            pltpu.VMEM((B, 64), _f32),
            pltpu.VMEM((B, 16), _f32),
            pltpu.VMEM((B,), _f32),
            pltpu.SemaphoreType.DMA,
            pltpu.SemaphoreType.DMA,
            pltpu.SemaphoreType.DMA,
            pltpu.SemaphoreType.DMA,
        ],
)(_sc_gat_body)


def _pool_body(acc_ref, den_ref, r4_ref, b2_ref, pb_ref, g16_ref, gf_ref,
               wg_ref,
               bg_ref, q1a_ref, q1b_ref, q1c_ref, q1b_b_ref, q2w_ref,
               q2b_ref, q3w_ref, q3b_ref, out_ref,
               sums, cnts, maxs):
    i = pl.program_id(0)

    @pl.when(i == 0)
    def _init():
        sums[...] = jnp.zeros((G, 64), _f32)
        cnts[...] = jnp.zeros((G, 64), _f32)
        maxs[...] = jnp.full((G, 64), -jnp.inf, _f32)

    num = acc_ref[...]
    den = jnp.dot(den_ref[...], r4_ref[...], preferred_element_type=_f32)
    h = _elu(num / (den + 1e-16) + b2_ref[...])

    bcol = jnp.dot(pb_ref[...], g16_ref[...], preferred_element_type=_f32)
    onef = jnp.ones((RB, 64), _f32)
    for g in range(G):
        m = bcol == float(g + 1)
        sums[g:g + 1, :] += jnp.sum(jnp.where(m, h, 0.0), axis=0,
                                    keepdims=True)
        cnts[g:g + 1, :] += jnp.sum(jnp.where(m, onef, 0.0), axis=0,
                                    keepdims=True)
        mg = jnp.max(jnp.where(m, h, -jnp.inf), axis=0, keepdims=True)
        maxs[g:g + 1, :] = jnp.maximum(maxs[g:g + 1, :], mg)

    @pl.when(i == NBLK - 1)
    def _fin():
        cnt = cnts[...]
        x_mean = sums[...] / jnp.maximum(cnt, 1.0)
        x_max = jnp.where(cnt > 0, maxs[...], 0.0)
        gf = jnp.maximum(jnp.dot(gf_ref[...], wg_ref[...],
                                 preferred_element_type=_f32)
                         + bg_ref[...], 0.0)
        q = (jnp.dot(x_mean, q1a_ref[...], preferred_element_type=_f32)
             + jnp.dot(x_max, q1b_ref[...], preferred_element_type=_f32)
             + jnp.dot(gf, q1c_ref[...], preferred_element_type=_f32)
             + q1b_b_ref[...])
        q = jnp.maximum(q, 0.0)
        q = jnp.maximum(jnp.dot(q, q2w_ref[...],
                                preferred_element_type=_f32) + q2b_ref[...],
                        0.0)
        out_ref[...] = (jnp.dot(q, q3w_ref[...], preferred_element_type=_f32)
                        + q3b_ref[...])


_pool = pl.pallas_call(
    _pool_body,
    grid=(NBLK,),
    in_specs=[
        pl.BlockSpec((RB, 64), lambda i: (i, 0)),
        pl.BlockSpec((RB, 4), lambda i: (i, 0)),
        pl.BlockSpec((4, 64), lambda i: (0, 0)),
        pl.BlockSpec((1, 64), lambda i: (0, 0)),
        pl.BlockSpec((RB, 16), lambda i: (i, 0)),
        pl.BlockSpec((16, 64), lambda i: (0, 0)),
        pl.BlockSpec((16, 128), lambda i: (0, 0)),
        pl.BlockSpec((128, 64), lambda i: (0, 0)),
        pl.BlockSpec((1, 64), lambda i: (0, 0)),
        pl.BlockSpec((64, 128), lambda i: (0, 0)),
        pl.BlockSpec((64, 128), lambda i: (0, 0)),
        pl.BlockSpec((64, 128), lambda i: (0, 0)),
        pl.BlockSpec((1, 128), lambda i: (0, 0)),
        pl.BlockSpec((128, 64), lambda i: (0, 0)),
        pl.BlockSpec((1, 64), lambda i: (0, 0)),
        pl.BlockSpec((64, 32), lambda i: (0, 0)),
        pl.BlockSpec((1, 32), lambda i: (0, 0)),
    ],
    out_specs=pl.BlockSpec((G, 32), lambda i: (0, 0)),
    out_shape=jax.ShapeDtypeStruct((G, 32), _f32),
    scratch_shapes=[
        pltpu.VMEM((G, 64), _f32),
        pltpu.VMEM((G, 64), _f32),
        pltpu.VMEM((G, 64), _f32),
    ],
)


def _att_mat(a_s, a_d):
    eye = jnp.eye(4, dtype=_f32)
    a_s = (a_s[:, :, None] * eye[:, None, :]).reshape(64, 4)
    a_d = (a_d[:, :, None] * eye[:, None, :]).reshape(64, 4)
    return jnp.concatenate([a_s, a_d], axis=1)


def kernel(x, edge_index, batch, global_features, W_emb, b_emb,
           W0, a_src0, a_dst0, b0, W1, a_src1, a_dst1, b1,
           W2, a_src2, a_dst2, b2, W_glob, b_glob,
           q1_W, q1_b, q2_W, q2_b, q3_W, q3_b):
    x_pad = jnp.pad(x, ((0, N_PAD - N), (0, 0)))
    src = edge_index[0]
    dst = edge_index[1]

    r4 = jnp.repeat(jnp.eye(4, dtype=_f32), 16, axis=1)
    pb = (jnp.pad(batch, (0, N_PAD - N), constant_values=G)[:, None]
          == jnp.arange(G)[None, :]).astype(_f32)
    g16 = jnp.broadcast_to(jnp.arange(1, G + 1, dtype=_f32)[:, None], (G, 64))
    gfp = jnp.pad(global_features, ((0, 0), (0, 125)))
    wgp = jnp.pad(W_glob, ((0, 125), (0, 0)))
    q3wp = jnp.pad(q3_W, ((0, 0), (0, 2)))
    q3bp = jnp.pad(q3_b, (0, 2))

    z = jnp.zeros((RPT, 64), _f32)
    z2 = jnp.zeros((RPT4, 16), _f32)

    hs, ed = _prep0(x_pad, W_emb, b_emb.reshape(1, 64), W0,
                    _att_mat(a_src0, a_dst0))
    acc, den2 = _sc_gat(hs, ed, src, dst, z, z2)
    hs, ed = _prepl(acc, den2.reshape(N_PAD, 4), r4, b0.reshape(1, 64), W1,
                    _att_mat(a_src1, a_dst1))
    acc, den2 = _sc_gat(hs, ed, src, dst, z, z2)
    hs, ed = _prepl(acc, den2.reshape(N_PAD, 4), r4, b1.reshape(1, 64), W2,
                    _att_mat(a_src2, a_dst2))
    acc, den2 = _sc_gat(hs, ed, src, dst, z, z2)

    q = _pool(acc, den2.reshape(N_PAD, 4), r4, b2.reshape(1, 64), pb, g16,
              gfp, wgp,
              b_glob.reshape(1, 64), q1_W[0:64], q1_W[64:128], q1_W[128:192],
              q1_b.reshape(1, 128), q2_W, q2_b.reshape(1, 64), q3wp,
              q3bp.reshape(1, 32))
    return q[:, 0:30]


# speedup vs baseline: 47.1417x; 1.2601x over previous
import functools

import jax
import jax.numpy as jnp
from jax import lax
from jax.experimental import pallas as pl
from jax.experimental.pallas import tpu as pltpu
from jax.experimental.pallas import tpu_sc as plsc

N = 50000
E = 800000
G = 16
NC = 2
NS = 16
LANES = 16
RPT = 1568
R = NS * RPT
N_PAD = NC * R
RB = 512
NBLK = N_PAD // RB
EPT = E // NS
B = 80
NCH = EPT // B
R4 = R // 4
RPT4 = RPT // 4

_f32 = jnp.float32


def _elu(z):
    return jnp.where(z > 0, z, jnp.exp(z) - 1.0)


def _prep0_body(x_ref, wemb_ref, bemb_ref, w_ref, aa_ref, hs_ref, ed_ref):
    h = _elu(jnp.dot(x_ref[...], wemb_ref[...],
                     preferred_element_type=_f32) + bemb_ref[...])
    hw = jnp.dot(h, w_ref[...], preferred_element_type=_f32)
    esed = jnp.dot(hw, aa_ref[...], preferred_element_type=_f32)
    hs_ref[...] = jnp.concatenate(
        [hw, esed[:, 0:4], jnp.zeros((RB, 12), _f32)], axis=1)
    ed_ref[...] = jnp.concatenate(
        [esed[:, 4:8], jnp.zeros((RB, 12), _f32)], axis=1)


_prep0 = pl.pallas_call(
    _prep0_body,
    grid=(NBLK,),
    in_specs=[
        pl.BlockSpec((RB, 128), lambda i: (i, 0)),
        pl.BlockSpec((128, 64), lambda i: (0, 0)),
        pl.BlockSpec((1, 64), lambda i: (0, 0)),
        pl.BlockSpec((64, 64), lambda i: (0, 0)),
        pl.BlockSpec((64, 8), lambda i: (0, 0)),
    ],
    out_specs=[
        pl.BlockSpec((RB, 80), lambda i: (i, 0)),
        pl.BlockSpec((RB, 16), lambda i: (i, 0)),
    ],
    out_shape=[
        jax.ShapeDtypeStruct((N_PAD, 80), _f32),
        jax.ShapeDtypeStruct((N_PAD, 16), _f32),
    ],
)


def _prepl_body(acc_ref, den_ref, r4_ref, bprev_ref, w_ref, aa_ref,
                hs_ref, ed_ref):
    num = acc_ref[...]
    den = jnp.dot(den_ref[...], r4_ref[...], preferred_element_type=_f32)
    h = _elu(num / (den + 1e-16) + bprev_ref[...])
    hw = jnp.dot(h, w_ref[...], preferred_element_type=_f32)
    esed = jnp.dot(hw, aa_ref[...], preferred_element_type=_f32)
    hs_ref[...] = jnp.concatenate(
        [hw, esed[:, 0:4], jnp.zeros((RB, 12), _f32)], axis=1)
    ed_ref[...] = jnp.concatenate(
        [esed[:, 4:8], jnp.zeros((RB, 12), _f32)], axis=1)


_prepl = pl.pallas_call(
    _prepl_body,
    grid=(NBLK,),
    in_specs=[
        pl.BlockSpec((RB, 64), lambda i: (i, 0)),
        pl.BlockSpec((RB, 4), lambda i: (i, 0)),
        pl.BlockSpec((4, 64), lambda i: (0, 0)),
        pl.BlockSpec((1, 64), lambda i: (0, 0)),
        pl.BlockSpec((64, 64), lambda i: (0, 0)),
        pl.BlockSpec((64, 8), lambda i: (0, 0)),
    ],
    out_specs=[
        pl.BlockSpec((RB, 80), lambda i: (i, 0)),
        pl.BlockSpec((RB, 16), lambda i: (i, 0)),
    ],
    out_shape=[
        jax.ShapeDtypeStruct((N_PAD, 80), _f32),
        jax.ShapeDtypeStruct((N_PAD, 16), _f32),
    ],
)


def _sc_gat_body(hs_hbm, ed_hbm, src_hbm, dst_hbm, z_hbm, z2_hbm,
         out_hbm, out2_hbm,
         acc, acc2, sidx0, didx0, sidx1, didx1, dloc, dloc2,
         hsb0, edb0, hsb1, edb1, contrib, contrib2, wbuf,
         semh0, seme0, semh1, seme1, semi0, semi1):
    c = lax.axis_index("c"); s = lax.axis_index("s")
    base = c * R
    pltpu.sync_copy(z_hbm, acc.at[pl.ds(s * RPT, RPT)])
    pltpu.sync_copy(z2_hbm, acc2.at[pl.ds(s * RPT4, RPT4)])
    plsc.subcore_barrier()
    lanes = lax.iota(jnp.int32, 16)
    kmask = jnp.where(lanes < 4, 1.0, 0.0)
    bufs = ((sidx0, didx0, hsb0, edb0, semh0, seme0, semi0),
            (sidx1, didx1, hsb1, edb1, semh1, seme1, semi1))

    def idxcopy(ch, bi):
        sidx, didx, hsb, edb, sh, se, si = bufs[bi]
        off = s * EPT + ch * B
        pltpu.async_copy(src_hbm.at[pl.ds(off, B)], sidx, si)
        pltpu.async_copy(dst_hbm.at[pl.ds(off, B)], didx, si)

    def idxwait(bi):
        sidx, didx, hsb, edb, sh, se, si = bufs[bi]
        pltpu.make_async_copy(src_hbm.at[pl.ds(0, B)], sidx, si).wait()
        pltpu.make_async_copy(dst_hbm.at[pl.ds(0, B)], didx, si).wait()

    def gather(bi):
        sidx, didx, hsb, edb, sh, se, si = bufs[bi]
        pltpu.async_copy(hs_hbm.at[sidx], hsb, sh)
        pltpu.async_copy(ed_hbm.at[didx], edb, se)

    def wait(bi):
        sidx, didx, hsb, edb, sh, se, si = bufs[bi]
        pltpu.make_async_copy(hs_hbm.at[sidx], hsb, sh).wait()
        pltpu.make_async_copy(ed_hbm.at[didx], edb, se).wait()

    def compute(bi):
        sidx, didx, hsb, edb, sh, se, si = bufs[bi]
        for j in range(B // 16):
            dv = didx[pl.ds(16 * j, 16)]
            own = (dv >= base) & (dv < base + R)
            dl = jnp.where(own, dv - base, 0)
            dloc[pl.ds(16 * j, 16)] = dl
            dloc2[pl.ds(16 * j, 16)] = lax.shift_right_logical(dl, 2)
            wbuf[pl.ds(16 * j, 16)] = jnp.where(own, 1.0, 0.0)
        def _grp(j, _):
            j16 = pl.multiple_of(j * 16, 16)
            mfv = wbuf[pl.ds(j16, 16)]
            dlv = dloc[pl.ds(j16, 16)]
            for jj in range(16):
                b = j16 + jj
                ev = hsb[b, pl.ds(64, 16)] + edb[b, pl.ds(0, 16)]
                ev = jnp.where(ev >= 0, ev, 0.2 * ev)
                wv = jnp.exp(ev) * (kmask * mfv[jj])
                for k in range(4):
                    contrib[b, pl.ds(16 * k, 16)] = (
                        hsb[b, pl.ds(16 * k, 16)] * wv[k])
                m4 = (dlv[jj] & 3) * 4
                den = jnp.where(lanes == m4, wv[0], 0.0)
                den = jnp.where(lanes == m4 + 1, wv[1], den)
                den = jnp.where(lanes == m4 + 2, wv[2], den)
                den = jnp.where(lanes == m4 + 3, wv[3], den)
                contrib2[b, pl.ds(0, 16)] = den
            return 0
        lax.fori_loop(0, B // 16, _grp, 0)
        pltpu.sync_copy(contrib, acc.at[dloc], add=True)
        pltpu.sync_copy(contrib2, acc2.at[dloc2], add=True)

    idxcopy(0, 0)
    idxwait(0)
    gather(0)
    idxcopy(1, 1)

    def pair(i, _):
        wait(0)
        idxwait(1)
        gather(1)
        idxcopy(2 * i + 2, 0)
        compute(0)
        wait(1)
        idxwait(0)
        gather(0)
        idxcopy(2 * i + 3, 1)
        compute(1)
        return 0
    lax.fori_loop(0, (NCH - 1) // 2, pair, 0)
    wait(0)
    idxwait(1)
    compute(0)

_sc_gat = functools.partial(
    pl.kernel,
    out_type=(jax.ShapeDtypeStruct((N_PAD, 64), _f32),
              jax.ShapeDtypeStruct((N_PAD // 4, 16), _f32)),
    mesh=plsc.VectorSubcoreMesh(
        core_axis_name="c", subcore_axis_name="s",
        num_cores=NC, num_subcores=NS),
    compiler_params=pltpu.CompilerParams(use_tc_tiling_on_sc=False),
            scratch_types=[
            pltpu.VMEM_SHARED((R, 64), _f32),
            pltpu.VMEM_SHARED((R4, 16), _f32),
            pltpu.VMEM((B,), jnp.int32),
            pltpu.VMEM((B,), jnp.int32),
            pltpu.VMEM((B,), jnp.int32),
            pltpu.VMEM((B,), jnp.int32),
            pltpu.VMEM((B,), jnp.int32),
            pltpu.VMEM((B,), jnp.int32),
            pltpu.VMEM((B, 80), _f32),
            pltpu.VMEM((B, 16), _f32),
            pltpu.VMEM((B, 80), _f32),
            pltpu.VMEM((B, 16), _f32),
            pltpu.VMEM((B, 64), _f32),
            pltpu.VMEM((B, 16), _f32),
            pltpu.VMEM((B,), _f32),
            pltpu.SemaphoreType.DMA,
            pltpu.SemaphoreType.DMA,
            pltpu.SemaphoreType.DMA,
            pltpu.SemaphoreType.DMA,
            pltpu.SemaphoreType.DMA,
            pltpu.SemaphoreType.DMA,
        ],
)(_sc_gat_body)


def _pool_body(acc_ref, den_ref, r4_ref, b2_ref, pb_ref, g16_ref, gf_ref,
               wg_ref,
               bg_ref, q1a_ref, q1b_ref, q1c_ref, q1b_b_ref, q2w_ref,
               q2b_ref, q3w_ref, q3b_ref, out_ref,
               sums, cnts, maxs):
    i = pl.program_id(0)

    @pl.when(i == 0)
    def _init():
        sums[...] = jnp.zeros((G, 64), _f32)
        cnts[...] = jnp.zeros((G, 64), _f32)
        maxs[...] = jnp.full((G, 64), -jnp.inf, _f32)

    num = acc_ref[...]
    den = jnp.dot(den_ref[...], r4_ref[...], preferred_element_type=_f32)
    h = _elu(num / (den + 1e-16) + b2_ref[...])

    bcol = jnp.dot(pb_ref[...], g16_ref[...], preferred_element_type=_f32)
    onef = jnp.ones((RB, 64), _f32)
    for g in range(G):
        m = bcol == float(g + 1)
        sums[g:g + 1, :] += jnp.sum(jnp.where(m, h, 0.0), axis=0,
                                    keepdims=True)
        cnts[g:g + 1, :] += jnp.sum(jnp.where(m, onef, 0.0), axis=0,
                                    keepdims=True)
        mg = jnp.max(jnp.where(m, h, -jnp.inf), axis=0, keepdims=True)
        maxs[g:g + 1, :] = jnp.maximum(maxs[g:g + 1, :], mg)

    @pl.when(i == NBLK - 1)
    def _fin():
        cnt = cnts[...]
        x_mean = sums[...] / jnp.maximum(cnt, 1.0)
        x_max = jnp.where(cnt > 0, maxs[...], 0.0)
        gf = jnp.maximum(jnp.dot(gf_ref[...], wg_ref[...],
                                 preferred_element_type=_f32)
                         + bg_ref[...], 0.0)
        q = (jnp.dot(x_mean, q1a_ref[...], preferred_element_type=_f32)
             + jnp.dot(x_max, q1b_ref[...], preferred_element_type=_f32)
             + jnp.dot(gf, q1c_ref[...], preferred_element_type=_f32)
             + q1b_b_ref[...])
        q = jnp.maximum(q, 0.0)
        q = jnp.maximum(jnp.dot(q, q2w_ref[...],
                                preferred_element_type=_f32) + q2b_ref[...],
                        0.0)
        out_ref[...] = (jnp.dot(q, q3w_ref[...], preferred_element_type=_f32)
                        + q3b_ref[...])


_pool = pl.pallas_call(
    _pool_body,
    grid=(NBLK,),
    in_specs=[
        pl.BlockSpec((RB, 64), lambda i: (i, 0)),
        pl.BlockSpec((RB, 4), lambda i: (i, 0)),
        pl.BlockSpec((4, 64), lambda i: (0, 0)),
        pl.BlockSpec((1, 64), lambda i: (0, 0)),
        pl.BlockSpec((RB, 16), lambda i: (i, 0)),
        pl.BlockSpec((16, 64), lambda i: (0, 0)),
        pl.BlockSpec((16, 128), lambda i: (0, 0)),
        pl.BlockSpec((128, 64), lambda i: (0, 0)),
        pl.BlockSpec((1, 64), lambda i: (0, 0)),
        pl.BlockSpec((64, 128), lambda i: (0, 0)),
        pl.BlockSpec((64, 128), lambda i: (0, 0)),
        pl.BlockSpec((64, 128), lambda i: (0, 0)),
        pl.BlockSpec((1, 128), lambda i: (0, 0)),
        pl.BlockSpec((128, 64), lambda i: (0, 0)),
        pl.BlockSpec((1, 64), lambda i: (0, 0)),
        pl.BlockSpec((64, 32), lambda i: (0, 0)),
        pl.BlockSpec((1, 32), lambda i: (0, 0)),
    ],
    out_specs=pl.BlockSpec((G, 32), lambda i: (0, 0)),
    out_shape=jax.ShapeDtypeStruct((G, 32), _f32),
    scratch_shapes=[
        pltpu.VMEM((G, 64), _f32),
        pltpu.VMEM((G, 64), _f32),
        pltpu.VMEM((G, 64), _f32),
    ],
)


def _att_mat(a_s, a_d):
    eye = jnp.eye(4, dtype=_f32)
    a_s = (a_s[:, :, None] * eye[:, None, :]).reshape(64, 4)
    a_d = (a_d[:, :, None] * eye[:, None, :]).reshape(64, 4)
    return jnp.concatenate([a_s, a_d], axis=1)


def kernel(x, edge_index, batch, global_features, W_emb, b_emb,
           W0, a_src0, a_dst0, b0, W1, a_src1, a_dst1, b1,
           W2, a_src2, a_dst2, b2, W_glob, b_glob,
           q1_W, q1_b, q2_W, q2_b, q3_W, q3_b):
    x_pad = jnp.pad(x, ((0, N_PAD - N), (0, 0)))
    src = jnp.pad(edge_index[0], (0, B))
    dst = jnp.pad(edge_index[1], (0, B))

    r4 = jnp.repeat(jnp.eye(4, dtype=_f32), 16, axis=1)
    pb = (jnp.pad(batch, (0, N_PAD - N), constant_values=G)[:, None]
          == jnp.arange(G)[None, :]).astype(_f32)
    g16 = jnp.broadcast_to(jnp.arange(1, G + 1, dtype=_f32)[:, None], (G, 64))
    gfp = jnp.pad(global_features, ((0, 0), (0, 125)))
    wgp = jnp.pad(W_glob, ((0, 125), (0, 0)))
    q3wp = jnp.pad(q3_W, ((0, 0), (0, 2)))
    q3bp = jnp.pad(q3_b, (0, 2))

    z = jnp.zeros((RPT, 64), _f32)
    z2 = jnp.zeros((RPT4, 16), _f32)

    hs, ed = _prep0(x_pad, W_emb, b_emb.reshape(1, 64), W0,
                    _att_mat(a_src0, a_dst0))
    acc, den2 = _sc_gat(hs, ed, src, dst, z, z2)
    hs, ed = _prepl(acc, den2.reshape(N_PAD, 4), r4, b0.reshape(1, 64), W1,
                    _att_mat(a_src1, a_dst1))
    acc, den2 = _sc_gat(hs, ed, src, dst, z, z2)
    hs, ed = _prepl(acc, den2.reshape(N_PAD, 4), r4, b1.reshape(1, 64), W2,
                    _att_mat(a_src2, a_dst2))
    acc, den2 = _sc_gat(hs, ed, src, dst, z, z2)

    q = _pool(acc, den2.reshape(N_PAD, 4), r4, b2.reshape(1, 64), pb, g16,
              gfp, wgp,
              b_glob.reshape(1, 64), q1_W[0:64], q1_W[64:128], q1_W[128:192],
              q1_b.reshape(1, 128), q2_W, q2_b.reshape(1, 64), q3wp,
              q3bp.reshape(1, 32))
    return q[:, 0:30]
